# baseline (speedup 1.0000x reference)
import functools

import jax
import jax.numpy as jnp
from jax import lax
from jax.experimental import pallas as pl
from jax.experimental.pallas import tpu as pltpu
from jax.experimental.pallas import tpu_sc as plsc

BATCH = 16384
EMBED = 32
SLAB = 4

_info = plsc.get_sparse_core_info()
_NC, _NS = _info.num_cores, _info.num_subcores
_NW = _NC * _NS
_BPW = BATCH // _NW
_CK = 128
_NCK = _BPW // _CK


def _sc_gather(user, item, t_mlp_u, t_mlp_i, t_gmf_u, t_gmf_i):
    out_sd = jax.ShapeDtypeStruct((BATCH, 4 * EMBED), jnp.float32)
    mesh = plsc.VectorSubcoreMesh(core_axis_name="c", subcore_axis_name="s")

    @functools.partial(
        pl.kernel,
        mesh=mesh,
        out_type=out_sd,
        compiler_params=pltpu.CompilerParams(needs_layout_passes=False),
        scratch_types=[
            pltpu.VMEM((_BPW,), jnp.int32),
            pltpu.VMEM((_BPW,), jnp.int32),
            pltpu.VMEM((_CK,), jnp.int32),
            pltpu.VMEM((_CK,), jnp.int32),
            pltpu.VMEM((_CK, SLAB * EMBED), jnp.float32),
            pltpu.VMEM((_BPW, 4 * EMBED), jnp.float32),
            pltpu.SemaphoreType.DMA,
        ],
    )
    def k(u_hbm, i_hbm, tmu, tmi, tgu, tgi, o_hbm,
          uidx, iidx, sidx, srow, slab, obuf, sem):
        wid = lax.axis_index("s") * _NC + lax.axis_index("c")
        base = wid * _BPW
        pltpu.sync_copy(u_hbm.at[pl.ds(base, _BPW)], uidx)
        pltpu.sync_copy(i_hbm.at[pl.ds(base, _BPW)], iidx)

        def do_chunk(c, idx_ref, tbl, col0):
            for v in range(_CK // 16):
                ivec = idx_ref[pl.ds(c * _CK + v * 16, 16)]
                sidx[pl.ds(v * 16, 16)] = lax.shift_right_logical(ivec, 2)
                srow[pl.ds(v * 16, 16)] = lax.bitwise_and(ivec, SLAB - 1) * EMBED
            pltpu.async_copy(tbl.at[sidx], slab, sem).wait()
            lanes = lax.iota(jnp.int32, 16)
            for v in range(_CK // 16):
                rowv = srow[pl.ds(v * 16, 16)]
                elem = lanes + v * 16
                drow = c * _CK + v * 16 + lanes
                for col in range(EMBED):
                    colv = lax.full_like(lanes, col)
                    vals = plsc.load_gather(slab, [elem, rowv + col])
                    plsc.store_scatter(obuf, [drow, colv + col0], vals)

        def chunk_loop(c, carry):
            do_chunk(c, uidx, tmu, 0)
            do_chunk(c, iidx, tmi, EMBED)
            do_chunk(c, uidx, tgu, 2 * EMBED)
            do_chunk(c, iidx, tgi, 3 * EMBED)
            return carry

        lax.fori_loop(0, _NCK, chunk_loop, 0)
        pltpu.sync_copy(obuf, o_hbm.at[pl.ds(base, _BPW)])

    return k(user, item, t_mlp_u, t_mlp_i, t_gmf_u, t_gmf_i)


_BM = 4096


def _tc_body(x, w1, b1, w2, b2, w3, b3, w4, b4, wog, wom, bo, out):
    xb = x[...]
    h = jnp.maximum(
        jnp.dot(xb[:, : 2 * EMBED], w1[...],
                preferred_element_type=jnp.float32) + b1[...], 0.0)
    h = jnp.maximum(
        jnp.dot(h, w2[...], preferred_element_type=jnp.float32) + b2[...], 0.0)
    h = jnp.maximum(
        jnp.dot(h, w3[...], preferred_element_type=jnp.float32) + b3[...], 0.0)
    mlp = jnp.dot(h, w4[...], preferred_element_type=jnp.float32) + b4[...]
    gmf = jnp.dot(xb[:, 2 * EMBED: 3 * EMBED] * xb[:, 3 * EMBED:], wog[...],
                  preferred_element_type=jnp.float32)
    out[...] = mlp * wom[...] + gmf + bo[...]


def _tc_mlp(x, w1, b1, w2, b2, w3, b3, w4, b4, wog, wom, bo):
    grid = (BATCH // _BM,)
    row = lambda i: (i, 0)
    rep = lambda i: (0, 0)

    def full(a):
        return pl.BlockSpec(a.shape, rep)

    return pl.pallas_call(
        _tc_body,
        grid=grid,
        in_specs=[
            pl.BlockSpec((_BM, 4 * EMBED), row),
            full(w1), full(b1), full(w2), full(b2),
            full(w3), full(b3), full(w4), full(b4),
            full(wog), full(wom), full(bo),
        ],
        out_specs=pl.BlockSpec((_BM, 1), row),
        out_shape=jax.ShapeDtypeStruct((BATCH, 1), jnp.float32),
    )(x, w1, b1, w2, b2, w3, b3, w4, b4, wog, wom, bo)


def kernel(user, item, user_embed_gmf, item_embed_gmf, user_embed_mlp,
           item_embed_mlp, W1, b1, W2, b2, W3, b3, W4, b4, Wo, bo):
    slabs = lambda t: t.reshape(t.shape[0] // SLAB, SLAB * EMBED)
    packed = _sc_gather(
        user.astype(jnp.int32), item.astype(jnp.int32),
        slabs(user_embed_mlp), slabs(item_embed_mlp),
        slabs(user_embed_gmf), slabs(item_embed_gmf))
    return _tc_mlp(
        packed,
        W1, b1.reshape(1, -1),
        W2, b2.reshape(1, -1), W3, b3.reshape(1, -1),
        W4, b4.reshape(1, 1),
        Wo[:EMBED], Wo[EMBED:], bo.reshape(1, 1))


# speedup vs baseline: 1.4844x; 1.4844x over previous
import functools

import jax
import jax.numpy as jnp
from jax import lax
from jax.experimental import pallas as pl
from jax.experimental.pallas import tpu as pltpu
from jax.experimental.pallas import tpu_sc as plsc

BATCH = 16384
EMBED = 32

_info = plsc.get_sparse_core_info()
_NC, _NS = _info.num_cores, _info.num_subcores
_NW = _NC * _NS
_BPW = BATCH // _NW
_FK = 32


def _sc_gather(user, item, t_mlp_u, t_mlp_i, t_gmf_u, t_gmf_i):
    out_sd = jax.ShapeDtypeStruct((BATCH, 4 * EMBED), jnp.float32)
    mesh = plsc.VectorSubcoreMesh(core_axis_name="c", subcore_axis_name="s")

    @functools.partial(
        pl.kernel,
        mesh=mesh,
        out_type=out_sd,
        scratch_types=[
            pltpu.VMEM((_BPW,), jnp.int32),
            pltpu.VMEM((_BPW,), jnp.int32),
            pltpu.VMEM((_BPW, 4 * EMBED), jnp.float32),
            pltpu.SemaphoreType.DMA,
            pltpu.SemaphoreType.DMA,
            pltpu.SemaphoreType.DMA,
            pltpu.SemaphoreType.DMA,
        ],
    )
    def k(u_hbm, i_hbm, tmu, tmi, tgu, tgi, o_hbm, uidx, iidx, buf,
          sem0, sem1, sem2, sem3):
        wid = lax.axis_index("s") * _NC + lax.axis_index("c")
        base = wid * _BPW
        pltpu.sync_copy(u_hbm.at[pl.ds(base, _BPW)], uidx)
        pltpu.sync_copy(i_hbm.at[pl.ds(base, _BPW)], iidx)

        def chunk(c, carry):
            b0 = c * _FK
            for v in range(_FK // 16):
                uvec = uidx[pl.ds(b0 + v * 16, 16)]
                ivec = iidx[pl.ds(b0 + v * 16, 16)]
                for j in range(16):
                    i = b0 + v * 16 + j
                    ru = uvec[j]
                    ri = ivec[j]
                    pltpu.async_copy(tmu.at[ru], buf.at[i, pl.ds(0, EMBED)], sem0)
                    pltpu.async_copy(tmi.at[ri], buf.at[i, pl.ds(EMBED, EMBED)], sem1)
                    pltpu.async_copy(tgu.at[ru], buf.at[i, pl.ds(2 * EMBED, EMBED)], sem2)
                    pltpu.async_copy(tgi.at[ri], buf.at[i, pl.ds(3 * EMBED, EMBED)], sem3)
            for j in range(_FK):
                i = b0 + j
                pltpu.make_async_copy(tmu.at[0], buf.at[i, pl.ds(0, EMBED)], sem0).wait()
                pltpu.make_async_copy(tmu.at[0], buf.at[i, pl.ds(EMBED, EMBED)], sem1).wait()
                pltpu.make_async_copy(tmu.at[0], buf.at[i, pl.ds(2 * EMBED, EMBED)], sem2).wait()
                pltpu.make_async_copy(tmu.at[0], buf.at[i, pl.ds(3 * EMBED, EMBED)], sem3).wait()
            return carry

        lax.fori_loop(0, _BPW // _FK, chunk, 0)
        pltpu.sync_copy(buf, o_hbm.at[pl.ds(base, _BPW)])

    return k(user, item, t_mlp_u, t_mlp_i, t_gmf_u, t_gmf_i)


_BM = 4096


def _tc_body(x, w1, b1, w2, b2, w3, b3, w4, b4, wog, wom, bo, out):
    xb = x[...]
    h = jnp.maximum(
        jnp.dot(xb[:, : 2 * EMBED], w1[...],
                preferred_element_type=jnp.float32) + b1[...], 0.0)
    h = jnp.maximum(
        jnp.dot(h, w2[...], preferred_element_type=jnp.float32) + b2[...], 0.0)
    h = jnp.maximum(
        jnp.dot(h, w3[...], preferred_element_type=jnp.float32) + b3[...], 0.0)
    mlp = jnp.dot(h, w4[...], preferred_element_type=jnp.float32) + b4[...]
    gmf = jnp.dot(xb[:, 2 * EMBED: 3 * EMBED] * xb[:, 3 * EMBED:], wog[...],
                  preferred_element_type=jnp.float32)
    out[...] = mlp * wom[...] + gmf + bo[...]


def _tc_mlp(x, w1, b1, w2, b2, w3, b3, w4, b4, wog, wom, bo):
    grid = (BATCH // _BM,)
    row = lambda i: (i, 0)
    rep = lambda i: (0, 0)

    def full(a):
        return pl.BlockSpec(a.shape, rep)

    return pl.pallas_call(
        _tc_body,
        grid=grid,
        in_specs=[
            pl.BlockSpec((_BM, 4 * EMBED), row),
            full(w1), full(b1), full(w2), full(b2),
            full(w3), full(b3), full(w4), full(b4),
            full(wog), full(wom), full(bo),
        ],
        out_specs=pl.BlockSpec((_BM, 1), row),
        out_shape=jax.ShapeDtypeStruct((BATCH, 1), jnp.float32),
    )(x, w1, b1, w2, b2, w3, b3, w4, b4, wog, wom, bo)


def kernel(user, item, user_embed_gmf, item_embed_gmf, user_embed_mlp,
           item_embed_mlp, W1, b1, W2, b2, W3, b3, W4, b4, Wo, bo):
    packed = _sc_gather(
        user.astype(jnp.int32), item.astype(jnp.int32),
        user_embed_mlp, item_embed_mlp, user_embed_gmf, item_embed_gmf)
    return _tc_mlp(
        packed,
        W1, b1.reshape(1, -1),
        W2, b2.reshape(1, -1), W3, b3.reshape(1, -1),
        W4, b4.reshape(1, 1),
        Wo[:EMBED], Wo[EMBED:], bo.reshape(1, 1))


# speedup vs baseline: 1.4886x; 1.0028x over previous
import functools

import jax
import jax.numpy as jnp
from jax import lax
from jax.experimental import pallas as pl
from jax.experimental.pallas import tpu as pltpu
from jax.experimental.pallas import tpu_sc as plsc

BATCH = 16384
EMBED = 32

_info = plsc.get_sparse_core_info()
_NC, _NS = _info.num_cores, _info.num_subcores
_NW = _NC * _NS
_BPW = BATCH // _NW
_FK = 32


def _sc_gather(user, item, t_mlp_u, t_mlp_i, t_gmf_u, t_gmf_i):
    out_sd = jax.ShapeDtypeStruct((BATCH, 4 * EMBED), jnp.float32)
    mesh = plsc.VectorSubcoreMesh(core_axis_name="c", subcore_axis_name="s")

    @functools.partial(
        pl.kernel,
        mesh=mesh,
        out_type=out_sd,
        scratch_types=[
            pltpu.VMEM((_BPW,), jnp.int32),
            pltpu.VMEM((_BPW,), jnp.int32),
            pltpu.VMEM((_BPW, 4 * EMBED), jnp.float32),
            pltpu.SemaphoreType.DMA,
        ],
    )
    def k(u_hbm, i_hbm, tmu, tmi, tgu, tgi, o_hbm, uidx, iidx, buf, sem):
        wid = lax.axis_index("s") * _NC + lax.axis_index("c")
        base = wid * _BPW
        pltpu.sync_copy(u_hbm.at[pl.ds(base, _BPW)], uidx)
        pltpu.sync_copy(i_hbm.at[pl.ds(base, _BPW)], iidx)

        def chunk(c, carry):
            b0 = c * _FK
            for v in range(_FK // 16):
                uvec = uidx[pl.ds(b0 + v * 16, 16)]
                ivec = iidx[pl.ds(b0 + v * 16, 16)]
                for j in range(16):
                    i = b0 + v * 16 + j
                    ru = uvec[j]
                    ri = ivec[j]
                    pltpu.async_copy(tmu.at[ru], buf.at[i, pl.ds(0, EMBED)], sem)
                    pltpu.async_copy(tmi.at[ri], buf.at[i, pl.ds(EMBED, EMBED)], sem)
                    pltpu.async_copy(tgu.at[ru], buf.at[i, pl.ds(2 * EMBED, EMBED)], sem)
                    pltpu.async_copy(tgi.at[ri], buf.at[i, pl.ds(3 * EMBED, EMBED)], sem)
            for j in range(_FK):
                i = b0 + j
                pltpu.make_async_copy(tmu.at[0], buf.at[i], sem).wait()
            return carry

        lax.fori_loop(0, _BPW // _FK, chunk, 0)
        pltpu.sync_copy(buf, o_hbm.at[pl.ds(base, _BPW)])

    return k(user, item, t_mlp_u, t_mlp_i, t_gmf_u, t_gmf_i)


_BM = 4096


def _tc_body(x, w1, b1, w2, b2, w3, b3, w4, b4, wog, wom, bo, out):
    xb = x[...]
    h = jnp.maximum(
        jnp.dot(xb[:, : 2 * EMBED], w1[...],
                preferred_element_type=jnp.float32) + b1[...], 0.0)
    h = jnp.maximum(
        jnp.dot(h, w2[...], preferred_element_type=jnp.float32) + b2[...], 0.0)
    h = jnp.maximum(
        jnp.dot(h, w3[...], preferred_element_type=jnp.float32) + b3[...], 0.0)
    mlp = jnp.dot(h, w4[...], preferred_element_type=jnp.float32) + b4[...]
    gmf = jnp.dot(xb[:, 2 * EMBED: 3 * EMBED] * xb[:, 3 * EMBED:], wog[...],
                  preferred_element_type=jnp.float32)
    out[...] = mlp * wom[...] + gmf + bo[...]


def _tc_mlp(x, w1, b1, w2, b2, w3, b3, w4, b4, wog, wom, bo):
    grid = (BATCH // _BM,)
    row = lambda i: (i, 0)
    rep = lambda i: (0, 0)

    def full(a):
        return pl.BlockSpec(a.shape, rep)

    return pl.pallas_call(
        _tc_body,
        grid=grid,
        in_specs=[
            pl.BlockSpec((_BM, 4 * EMBED), row),
            full(w1), full(b1), full(w2), full(b2),
            full(w3), full(b3), full(w4), full(b4),
            full(wog), full(wom), full(bo),
        ],
        out_specs=pl.BlockSpec((_BM, 1), row),
        out_shape=jax.ShapeDtypeStruct((BATCH, 1), jnp.float32),
    )(x, w1, b1, w2, b2, w3, b3, w4, b4, wog, wom, bo)


def kernel(user, item, user_embed_gmf, item_embed_gmf, user_embed_mlp,
           item_embed_mlp, W1, b1, W2, b2, W3, b3, W4, b4, Wo, bo):
    packed = _sc_gather(
        user.astype(jnp.int32), item.astype(jnp.int32),
        user_embed_mlp, item_embed_mlp, user_embed_gmf, item_embed_gmf)
    return _tc_mlp(
        packed,
        W1, b1.reshape(1, -1),
        W2, b2.reshape(1, -1), W3, b3.reshape(1, -1),
        W4, b4.reshape(1, 1),
        Wo[:EMBED], Wo[EMBED:], bo.reshape(1, 1))
